# Initial kernel scaffold; baseline (speedup 1.0000x reference)
import jax
import jax.numpy as jnp
from jax.experimental import pallas as pl


def kernel(z_e, codebook):
    raise NotImplementedError("write your pallas kernel here")


# speedup vs baseline: 2.2292x; 2.2292x over previous
import functools

import jax
import jax.numpy as jnp
from jax.experimental import pallas as pl

_NUM_EMB = 512
_EMB_DIM = 256
_COMMIT = 0.25


def _vq_block(z_ref, zsq_ref, cb_ref, csq_ref, zq_ref, idx_ref, loss_ref):
    t = z_ref.shape[0]
    zb = z_ref[...]
    dot = jax.lax.dot_general(
        zb, cb_ref[...], (((1,), (1,)), ((), ())),
        preferred_element_type=jnp.float32)
    d = zsq_ref[...] + csq_ref[...] - 2.0 * dot
    iota = jax.lax.broadcasted_iota(jnp.int32, (t, _NUM_EMB), 1)
    m = jnp.min(d, axis=1, keepdims=True)
    idx = jnp.min(jnp.where(d == m, iota, _NUM_EMB), axis=1)
    onehot = (iota == idx[:, None]).astype(jnp.float32)
    zq = jnp.dot(onehot, cb_ref[...],
                 preferred_element_type=jnp.float32)
    diff = zq - zb
    part = jnp.sum(diff * diff).reshape(1, 1)
    zq_ref[...] = zb + (zq - zb)
    idx_ref[0, 0] = idx

    @pl.when(pl.program_id(0) == 0)
    def _init():
        loss_ref[...] = jnp.zeros((1, 1), jnp.float32)

    loss_ref[...] += part


@functools.partial(jax.jit, static_argnames=("t_block",))
def _vq(z_e, codebook, t_block=2048):
    b, c, d0, d1, d2 = z_e.shape
    ntok = b * d0 * d1 * d2
    nblk = ntok // t_block

    z_e_p = jnp.transpose(z_e, (0, 2, 3, 4, 1))
    flat_z = z_e_p.reshape(-1, c)
    zsq = jnp.sum(flat_z ** 2, axis=1, keepdims=True)
    csq = jnp.sum(codebook ** 2, axis=1, keepdims=True).T

    zq_flat, idx3, loss = pl.pallas_call(
        _vq_block,
        grid=(nblk,),
        in_specs=[
            pl.BlockSpec((t_block, c), lambda i: (i, 0)),
            pl.BlockSpec((t_block, 1), lambda i: (i, 0)),
            pl.BlockSpec((_NUM_EMB, _EMB_DIM), lambda i: (0, 0)),
            pl.BlockSpec((1, _NUM_EMB), lambda i: (0, 0)),
        ],
        out_specs=[
            pl.BlockSpec((t_block, c), lambda i: (i, 0)),
            pl.BlockSpec((1, 1, t_block), lambda i: (i, 0, 0)),
            pl.BlockSpec((1, 1), lambda i: (0, 0)),
        ],
        out_shape=[
            jax.ShapeDtypeStruct((ntok, c), jnp.float32),
            jax.ShapeDtypeStruct((nblk, 1, t_block), jnp.int32),
            jax.ShapeDtypeStruct((1, 1), jnp.float32),
        ],
    )(flat_z, zsq, codebook, csq)

    z_q = jnp.transpose(zq_flat.reshape(b, d0, d1, d2, c), (0, 4, 1, 2, 3))
    indices = idx3.reshape(b, d0, d1, d2)
    vq_loss = loss[0, 0] * (1.0 + _COMMIT) / (ntok * c)
    return z_q, vq_loss, indices


def kernel(z_e, codebook):
    return _vq(z_e, codebook)
